# Initial kernel scaffold; baseline (speedup 1.0000x reference)
import jax
import jax.numpy as jnp
from jax.experimental import pallas as pl


def kernel(x, W_q, keys, down_embed, up_embed):
    raise NotImplementedError("write your pallas kernel here")


# speedup vs baseline: 2.1648x; 2.1648x over previous
import functools

import jax
import jax.numpy as jnp
import numpy as np
from jax import lax
from jax.experimental import pallas as pl
from jax.experimental.pallas import tpu as pltpu
from jax.experimental.pallas import tpu_sc as plsc

N_DIM = 1024
NUM_HEADS = 8
NUM_EXPERTS = 4096
NUM_KEYS = 64
TOPK = 8
KEY_DIM = 512
N_TOKENS = 2048
PICKS = NUM_HEADS * TOPK

TB = 256


def _top8(s, width):
    iota = lax.broadcasted_iota(jnp.int32, s.shape, 1)
    vals, idxs = [], []
    for _ in range(TOPK):
        m = jnp.max(s, axis=1, keepdims=True)
        sel = jnp.min(jnp.where(s == m, iota, width), axis=1, keepdims=True)
        vals.append(m)
        idxs.append(sel)
        s = jnp.where(iota == sel, -1e30, s)
    return jnp.concatenate(vals, axis=1), jnp.concatenate(idxs, axis=1)


def _route_body(x_ref, wq_ref, keys_ref, idx_ref, w_ref):
    q = lax.dot_general(
        x_ref[...], wq_ref[...], (((1,), (0,)), ((), ())),
        preferred_element_type=jnp.float32)
    qb = q.astype(jnp.bfloat16)
    for h in range(NUM_HEADS):
        sx_s = lax.dot_general(
            qb[:, h * KEY_DIM:(h + 1) * KEY_DIM], keys_ref[0, h],
            (((1,), (1,)), ((), ())),
            preferred_element_type=jnp.float32)
        sy_s = lax.dot_general(
            qb[:, NUM_HEADS * KEY_DIM + h * KEY_DIM:
               NUM_HEADS * KEY_DIM + (h + 1) * KEY_DIM], keys_ref[1, h],
            (((1,), (1,)), ((), ())),
            preferred_element_type=jnp.float32)
        sx, ix = _top8(sx_s, NUM_KEYS)
        sy, iy = _top8(sy_s, NUM_KEYS)
        comb = jnp.concatenate(
            [sx[:, j:j + 1] + sy for j in range(TOPK)], axis=1)
        s8, pos = _top8(comb, NUM_KEYS)
        pj = pos // TOPK
        plo = pos % TOPK
        ixe = jnp.zeros_like(pos)
        iye = jnp.zeros_like(pos)
        for j in range(TOPK):
            ixe = jnp.where(pj == j, ix[:, j:j + 1], ixe)
            iye = jnp.where(plo == j, iy[:, j:j + 1], iye)
        idx = ixe * NUM_KEYS + iye + h * NUM_EXPERTS
        m = jnp.max(s8, axis=1, keepdims=True)
        e = jnp.exp(s8 - m)
        w = e / jnp.sum(e, axis=1, keepdims=True)
        idx_ref[:, h, :] = idx
        w_ref[:, h, :] = w


def _route(x2d_bf16, W_q_bf16, keys_t_bf16):
    return pl.pallas_call(
        _route_body,
        grid=(N_TOKENS // TB,),
        in_specs=[
            pl.BlockSpec((TB, N_DIM), lambda i: (i, 0)),
            pl.BlockSpec((N_DIM, 2 * NUM_HEADS * KEY_DIM), lambda i: (0, 0)),
            pl.BlockSpec((2, NUM_HEADS, NUM_KEYS, KEY_DIM), lambda i: (0, 0, 0, 0)),
        ],
        out_specs=[
            pl.BlockSpec((TB, NUM_HEADS, TOPK), lambda i: (i, 0, 0)),
            pl.BlockSpec((TB, NUM_HEADS, TOPK), lambda i: (i, 0, 0)),
        ],
        out_shape=[
            jax.ShapeDtypeStruct((N_TOKENS, NUM_HEADS, TOPK), jnp.int32),
            jax.ShapeDtypeStruct((N_TOKENS, NUM_HEADS, TOPK), jnp.float32),
        ],
    )(x2d_bf16, W_q_bf16, keys_t_bf16)


_NC = 2
_NS = 16
_NW = _NC * _NS
_TPW = N_TOKENS // _NW
_CH = 32
_NCHUNK = PICKS // _CH

_SQRT1_2 = np.float32(0.7071067811865476)


def _gelu16(h):
    v = h * _SQRT1_2
    z = jnp.abs(v)
    t = 1.0 / (1.0 + 0.3275911 * z)
    poly = ((((1.061405429 * t - 1.453152027) * t + 1.421413741) * t
             - 0.284496736) * t + 0.254829592) * t
    erfp = 1.0 - poly * jnp.exp(-z * z)
    erfv = jnp.where(v >= 0, erfp, -erfp)
    return 0.5 * h * (1.0 + erfv)


def _sc_body(x_hbm, idx_hbm, w_hbm, down_hbm, up_hbm, out_hbm,
             x_v, idx_v, w_v, g_v, rows_v, out_v, sem):
    wid = lax.axis_index("s") * _NC + lax.axis_index("c")
    t0 = wid * _TPW

    def token_body(ti, carry):
        t = t0 + ti
        pltpu.sync_copy(x_hbm.at[t], x_v)
        pltpu.sync_copy(idx_hbm.at[t], idx_v)
        pltpu.sync_copy(w_hbm.at[t], w_v)

        iota16 = lax.iota(jnp.int32, 16)
        for c in range(_NCHUNK):
            pltpu.async_copy(
                down_hbm.at[idx_v.at[pl.ds(c * _CH, _CH)]], rows_v, sem
            ).wait()
            for rg in range(_CH // 16):
                def row_body(r, h16):
                    row = rg * 16 + r

                    def dot_body(cc, acc):
                        return acc + (rows_v[row, pl.ds(cc * 16, 16)]
                                      * x_v[pl.ds(cc * 16, 16)])
                    acc = lax.fori_loop(0, N_DIM // 16, dot_body,
                                        jnp.zeros((16,), jnp.float32))
                    s = jnp.sum(acc)
                    return jnp.where(iota16 == r, s, h16)
                h16 = lax.fori_loop(0, 16, row_body,
                                    jnp.zeros((16,), jnp.float32))
                j = c * (_CH // 16) + rg
                g_v[pl.ds(j * 16, 16)] = (
                    _gelu16(h16) * w_v[pl.ds(j * 16, 16)])

        def zero_body(cc, _):
            out_v[pl.ds(cc * 16, 16)] = jnp.zeros((16,), jnp.float32)
            return 0
        lax.fori_loop(0, N_DIM // 16, zero_body, 0)

        for c in range(_NCHUNK):
            pltpu.async_copy(
                up_hbm.at[idx_v.at[pl.ds(c * _CH, _CH)]], rows_v, sem
            ).wait()

            def urow_body(r, _):
                rg16 = g_v[pl.ds(c * _CH + (r // 16) * 16, 16)]
                s = jnp.sum(jnp.where(iota16 == (r % 16), rg16, 0.0))
                gb = lax.broadcast(s, (16,))

                def chunk_body(cc, _):
                    sl = pl.ds(cc * 16, 16)
                    out_v[sl] = out_v[sl] + rows_v[r, sl] * gb
                    return 0
                lax.fori_loop(0, N_DIM // 16, chunk_body, 0)
                return 0
            lax.fori_loop(0, _CH, urow_body, 0)

        pltpu.sync_copy(out_v, out_hbm.at[t])
        return carry

    lax.fori_loop(0, _TPW, token_body, 0)


def _sc_combine(x2d, idx, w, down_embed, up_embed):
    mesh = plsc.VectorSubcoreMesh(core_axis_name="c", subcore_axis_name="s")
    f = pl.kernel(
        _sc_body,
        out_type=jax.ShapeDtypeStruct((N_TOKENS, N_DIM), jnp.float32),
        mesh=mesh,
        compiler_params=pltpu.CompilerParams(needs_layout_passes=False),
        scratch_types=[
            pltpu.VMEM((N_DIM,), jnp.float32),
            pltpu.VMEM((PICKS,), jnp.int32),
            pltpu.VMEM((PICKS,), jnp.float32),
            pltpu.VMEM((PICKS,), jnp.float32),
            pltpu.VMEM((_CH, N_DIM), jnp.float32),
            pltpu.VMEM((N_DIM,), jnp.float32),
            pltpu.SemaphoreType.DMA,
        ],
    )
    return f(x2d, idx, w, down_embed, up_embed)


def kernel(x, W_q, keys, down_embed, up_embed):
    b, n, d = x.shape
    x2d = x.reshape(n, d)
    keys_t = jnp.transpose(keys, (2, 0, 1, 3))
    idx, w = _route(x2d.astype(jnp.bfloat16), W_q.astype(jnp.bfloat16),
                    keys_t.astype(jnp.bfloat16))
    idx = idx.reshape(N_TOKENS, PICKS)
    w = w.reshape(N_TOKENS, PICKS)
    out = _sc_combine(x2d, idx, w, down_embed, up_embed)
    return out.reshape(b, n, d)


# speedup vs baseline: 2.4009x; 1.1090x over previous
import functools

import jax
import jax.numpy as jnp
import numpy as np
from jax import lax
from jax.experimental import pallas as pl
from jax.experimental.pallas import tpu as pltpu
from jax.experimental.pallas import tpu_sc as plsc

N_DIM = 1024
NUM_HEADS = 8
NUM_EXPERTS = 4096
NUM_KEYS = 64
TOPK = 8
KEY_DIM = 512
N_TOKENS = 2048
PICKS = NUM_HEADS * TOPK

TB = 256


def _top8(s, width):
    iota = lax.broadcasted_iota(jnp.int32, s.shape, 1)
    vals, idxs = [], []
    for _ in range(TOPK):
        m = jnp.max(s, axis=1, keepdims=True)
        sel = jnp.min(jnp.where(s == m, iota, width), axis=1, keepdims=True)
        vals.append(m)
        idxs.append(sel)
        s = jnp.where(iota == sel, -1e30, s)
    return jnp.concatenate(vals, axis=1), jnp.concatenate(idxs, axis=1)


def _route_body(x_ref, wq_ref, keys_ref, idx_ref, w_ref):
    q = lax.dot_general(
        x_ref[...], wq_ref[...], (((1,), (0,)), ((), ())),
        preferred_element_type=jnp.float32)
    qb = q.astype(jnp.bfloat16)
    for h in range(NUM_HEADS):
        sx_s = lax.dot_general(
            qb[:, h * KEY_DIM:(h + 1) * KEY_DIM], keys_ref[0, h],
            (((1,), (1,)), ((), ())),
            preferred_element_type=jnp.float32)
        sy_s = lax.dot_general(
            qb[:, NUM_HEADS * KEY_DIM + h * KEY_DIM:
               NUM_HEADS * KEY_DIM + (h + 1) * KEY_DIM], keys_ref[1, h],
            (((1,), (1,)), ((), ())),
            preferred_element_type=jnp.float32)
        sx, ix = _top8(sx_s, NUM_KEYS)
        sy, iy = _top8(sy_s, NUM_KEYS)
        comb = jnp.concatenate(
            [sx[:, j:j + 1] + sy for j in range(TOPK)], axis=1)
        s8, pos = _top8(comb, NUM_KEYS)
        pj = pos // TOPK
        plo = pos % TOPK
        ixe = jnp.zeros_like(pos)
        iye = jnp.zeros_like(pos)
        for j in range(TOPK):
            ixe = jnp.where(pj == j, ix[:, j:j + 1], ixe)
            iye = jnp.where(plo == j, iy[:, j:j + 1], iye)
        idx = ixe * NUM_KEYS + iye + h * NUM_EXPERTS
        m = jnp.max(s8, axis=1, keepdims=True)
        e = jnp.exp(s8 - m)
        w = e / jnp.sum(e, axis=1, keepdims=True)
        idx_ref[:, h, :] = idx
        w_ref[:, h, :] = w


def _route(x2d_bf16, W_q_bf16, keys_t_bf16):
    return pl.pallas_call(
        _route_body,
        grid=(N_TOKENS // TB,),
        in_specs=[
            pl.BlockSpec((TB, N_DIM), lambda i: (i, 0)),
            pl.BlockSpec((N_DIM, 2 * NUM_HEADS * KEY_DIM), lambda i: (0, 0)),
            pl.BlockSpec((2, NUM_HEADS, NUM_KEYS, KEY_DIM), lambda i: (0, 0, 0, 0)),
        ],
        out_specs=[
            pl.BlockSpec((TB, NUM_HEADS, TOPK), lambda i: (i, 0, 0)),
            pl.BlockSpec((TB, NUM_HEADS, TOPK), lambda i: (i, 0, 0)),
        ],
        out_shape=[
            jax.ShapeDtypeStruct((N_TOKENS, NUM_HEADS, TOPK), jnp.int32),
            jax.ShapeDtypeStruct((N_TOKENS, NUM_HEADS, TOPK), jnp.float32),
        ],
    )(x2d_bf16, W_q_bf16, keys_t_bf16)


_NC = 2
_NS = 16
_NW = _NC * _NS
_TPW = N_TOKENS // _NW
_CH = 32
_NCHUNK = PICKS // _CH

_SQRT1_2 = np.float32(0.7071067811865476)


def _gelu16(h):
    v = h * _SQRT1_2
    z = jnp.abs(v)
    t = 1.0 / (1.0 + 0.3275911 * z)
    poly = ((((1.061405429 * t - 1.453152027) * t + 1.421413741) * t
             - 0.284496736) * t + 0.254829592) * t
    erfp = 1.0 - poly * jnp.exp(-z * z)
    erfv = jnp.where(v >= 0, erfp, -erfp)
    return 0.5 * h * (1.0 + erfv)


def _sc_body(x_hbm, idx_hbm, w_hbm, down_hbm, up_hbm, out_hbm,
             x_v, idx_v, w_v, g_v, rows_a, rows_b, out_v, sem_a, sem_b):
    wid = lax.axis_index("s") * _NC + lax.axis_index("c")
    t0 = wid * _TPW

    pltpu.sync_copy(idx_hbm.at[pl.ds(t0, _TPW)], idx_v)
    pltpu.sync_copy(w_hbm.at[pl.ds(t0, _TPW)], w_v)

    def token_body(ti, carry):
        t = t0 + ti
        cp_a = pltpu.make_async_copy(
            down_hbm.at[idx_v.at[ti, pl.ds(0, _CH)]], rows_a, sem_a)
        cp_a.start()
        cp_b = pltpu.make_async_copy(
            down_hbm.at[idx_v.at[ti, pl.ds(_CH, _CH)]], rows_b, sem_b)
        cp_b.start()
        pltpu.sync_copy(x_hbm.at[t], x_v)

        iota16 = lax.iota(jnp.int32, 16)

        def down_chunk(c, rows_v):
            for rg in range(_CH // 16):
                def row_body(r, h16):
                    row = rg * 16 + r

                    def dot_body(cc, acc):
                        return acc + (rows_v[row, pl.ds(cc * 16, 16)]
                                      * x_v[pl.ds(cc * 16, 16)])
                    acc = lax.fori_loop(0, N_DIM // 16, dot_body,
                                        jnp.zeros((16,), jnp.float32))
                    s = jnp.sum(acc)
                    return jnp.where(iota16 == r, s, h16)
                h16 = lax.fori_loop(0, 16, row_body,
                                    jnp.zeros((16,), jnp.float32))
                j = c * (_CH // 16) + rg
                g_v[pl.ds(j * 16, 16)] = (
                    _gelu16(h16) * w_v[ti, pl.ds(j * 16, 16)])

        cp_a.wait()
        down_chunk(0, rows_a)
        cp_b.wait()
        up_a = pltpu.make_async_copy(
            up_hbm.at[idx_v.at[ti, pl.ds(0, _CH)]], rows_a, sem_a)
        up_a.start()
        down_chunk(1, rows_b)
        up_b = pltpu.make_async_copy(
            up_hbm.at[idx_v.at[ti, pl.ds(_CH, _CH)]], rows_b, sem_b)
        up_b.start()

        def zero_body(cc, _):
            out_v[pl.ds(cc * 16, 16)] = jnp.zeros((16,), jnp.float32)
            return 0
        lax.fori_loop(0, N_DIM // 16, zero_body, 0)

        def up_chunk(c, rows_v):
            def urow_body(r, _):
                rg16 = g_v[pl.ds(c * _CH + (r // 16) * 16, 16)]
                s = jnp.sum(jnp.where(iota16 == (r % 16), rg16, 0.0))
                gb = lax.broadcast(s, (16,))

                def chunk_body(cc, _):
                    sl = pl.ds(cc * 16, 16)
                    out_v[sl] = out_v[sl] + rows_v[r, sl] * gb
                    return 0
                lax.fori_loop(0, N_DIM // 16, chunk_body, 0)
                return 0
            lax.fori_loop(0, _CH, urow_body, 0)

        up_a.wait()
        up_chunk(0, rows_a)
        up_b.wait()
        up_chunk(1, rows_b)

        pltpu.sync_copy(out_v, out_hbm.at[t])
        return carry

    lax.fori_loop(0, _TPW, token_body, 0)


def _sc_combine(x2d, idx, w, down_embed, up_embed):
    mesh = plsc.VectorSubcoreMesh(core_axis_name="c", subcore_axis_name="s")
    f = pl.kernel(
        _sc_body,
        out_type=jax.ShapeDtypeStruct((N_TOKENS, N_DIM), jnp.float32),
        mesh=mesh,
        compiler_params=pltpu.CompilerParams(needs_layout_passes=False),
        scratch_types=[
            pltpu.VMEM((N_DIM,), jnp.float32),
            pltpu.VMEM((_TPW, PICKS), jnp.int32),
            pltpu.VMEM((_TPW, PICKS), jnp.float32),
            pltpu.VMEM((PICKS,), jnp.float32),
            pltpu.VMEM((_CH, N_DIM), jnp.float32),
            pltpu.VMEM((_CH, N_DIM), jnp.float32),
            pltpu.VMEM((N_DIM,), jnp.float32),
            pltpu.SemaphoreType.DMA,
            pltpu.SemaphoreType.DMA,
        ],
    )
    return f(x2d, idx, w, down_embed, up_embed)


def kernel(x, W_q, keys, down_embed, up_embed):
    b, n, d = x.shape
    x2d = x.reshape(n, d)
    keys_t = jnp.transpose(keys, (2, 0, 1, 3))
    idx, w = _route(x2d.astype(jnp.bfloat16), W_q.astype(jnp.bfloat16),
                    keys_t.astype(jnp.bfloat16))
    idx = idx.reshape(N_TOKENS, PICKS)
    w = w.reshape(N_TOKENS, PICKS)
    out = _sc_combine(x2d, idx, w, down_embed, up_embed)
    return out.reshape(b, n, d)


# speedup vs baseline: 2.9352x; 1.2225x over previous
import functools

import jax
import jax.numpy as jnp
import numpy as np
from jax import lax
from jax.experimental import pallas as pl
from jax.experimental.pallas import tpu as pltpu
from jax.experimental.pallas import tpu_sc as plsc

N_DIM = 1024
NUM_HEADS = 8
NUM_EXPERTS = 4096
NUM_KEYS = 64
TOPK = 8
KEY_DIM = 512
N_TOKENS = 2048
PICKS = NUM_HEADS * TOPK

TB = 256


def _top8(s, width):
    iota = lax.broadcasted_iota(jnp.int32, s.shape, 1)
    vals, idxs = [], []
    for _ in range(TOPK):
        m = jnp.max(s, axis=1, keepdims=True)
        sel = jnp.min(jnp.where(s == m, iota, width), axis=1, keepdims=True)
        vals.append(m)
        idxs.append(sel)
        s = jnp.where(iota == sel, -1e30, s)
    return jnp.concatenate(vals, axis=1), jnp.concatenate(idxs, axis=1)


def _route_body(x_ref, wq_ref, keys_ref, idx_ref, w_ref):
    q = lax.dot_general(
        x_ref[...], wq_ref[...], (((1,), (0,)), ((), ())),
        preferred_element_type=jnp.float32)
    qb = q.astype(jnp.bfloat16)
    for h in range(NUM_HEADS):
        sx_s = lax.dot_general(
            qb[:, h * KEY_DIM:(h + 1) * KEY_DIM], keys_ref[0, h],
            (((1,), (1,)), ((), ())),
            preferred_element_type=jnp.float32)
        sy_s = lax.dot_general(
            qb[:, NUM_HEADS * KEY_DIM + h * KEY_DIM:
               NUM_HEADS * KEY_DIM + (h + 1) * KEY_DIM], keys_ref[1, h],
            (((1,), (1,)), ((), ())),
            preferred_element_type=jnp.float32)
        sx, ix = _top8(sx_s, NUM_KEYS)
        sy, iy = _top8(sy_s, NUM_KEYS)
        comb = jnp.concatenate(
            [sx[:, j:j + 1] + sy for j in range(TOPK)], axis=1)
        s8, pos = _top8(comb, NUM_KEYS)
        pj = pos // TOPK
        plo = pos % TOPK
        ixe = jnp.zeros_like(pos)
        iye = jnp.zeros_like(pos)
        for j in range(TOPK):
            ixe = jnp.where(pj == j, ix[:, j:j + 1], ixe)
            iye = jnp.where(plo == j, iy[:, j:j + 1], iye)
        idx = ixe * NUM_KEYS + iye + h * NUM_EXPERTS
        m = jnp.max(s8, axis=1, keepdims=True)
        e = jnp.exp(s8 - m)
        w = e / jnp.sum(e, axis=1, keepdims=True)
        idx_ref[:, h, :] = idx
        w_ref[:, h, :] = w


def _route(x2d_bf16, W_q_bf16, keys_t_bf16):
    return pl.pallas_call(
        _route_body,
        grid=(N_TOKENS // TB,),
        in_specs=[
            pl.BlockSpec((TB, N_DIM), lambda i: (i, 0)),
            pl.BlockSpec((N_DIM, 2 * NUM_HEADS * KEY_DIM), lambda i: (0, 0)),
            pl.BlockSpec((2, NUM_HEADS, NUM_KEYS, KEY_DIM), lambda i: (0, 0, 0, 0)),
        ],
        out_specs=[
            pl.BlockSpec((TB, NUM_HEADS, TOPK), lambda i: (i, 0, 0)),
            pl.BlockSpec((TB, NUM_HEADS, TOPK), lambda i: (i, 0, 0)),
        ],
        out_shape=[
            jax.ShapeDtypeStruct((N_TOKENS, NUM_HEADS, TOPK), jnp.int32),
            jax.ShapeDtypeStruct((N_TOKENS, NUM_HEADS, TOPK), jnp.float32),
        ],
    )(x2d_bf16, W_q_bf16, keys_t_bf16)


_NC = 2
_NS = 16
_NW = _NC * _NS
_TPW = N_TOKENS // _NW
_CH = 32
_NCHUNK = PICKS // _CH

_SQRT1_2 = np.float32(0.7071067811865476)


def _gelu16(h):
    v = h * _SQRT1_2
    z = jnp.abs(v)
    t = 1.0 / (1.0 + 0.3275911 * z)
    poly = ((((1.061405429 * t - 1.453152027) * t + 1.421413741) * t
             - 0.284496736) * t + 0.254829592) * t
    erfp = 1.0 - poly * jnp.exp(-z * z)
    erfv = jnp.where(v >= 0, erfp, -erfp)
    return 0.5 * h * (1.0 + erfv)


def _sc_body(x_hbm, idx_hbm, w_hbm, down_hbm, up_hbm, out_hbm,
             x_v, idx_v, w_v, g_v, rows_a, rows_b, out_v, sem_a, sem_b):
    wid = lax.axis_index("s") * _NC + lax.axis_index("c")
    t0 = wid * _TPW

    pltpu.sync_copy(idx_hbm.at[pl.ds(t0, _TPW)], idx_v)
    pltpu.sync_copy(w_hbm.at[pl.ds(t0, _TPW)], w_v)

    def token_body(ti, carry):
        t = t0 + ti
        cp_a = pltpu.make_async_copy(
            down_hbm.at[idx_v.at[ti, pl.ds(0, _CH)]], rows_a, sem_a)
        cp_a.start()
        cp_b = pltpu.make_async_copy(
            down_hbm.at[idx_v.at[ti, pl.ds(_CH, _CH)]], rows_b, sem_b)
        cp_b.start()
        pltpu.sync_copy(x_hbm.at[t], x_v)

        iota16 = lax.iota(jnp.int32, 16)

        def down_chunk(c, rows_v):
            for rg in range(_CH // 16):
                def row_body(r, h16):
                    row = rg * 16 + r

                    def dot_body(cc, acc):
                        return acc + (rows_v[row, pl.ds(cc * 16, 16)]
                                      * x_v[pl.ds(cc * 16, 16)])
                    acc = lax.fori_loop(0, N_DIM // 16, dot_body,
                                        jnp.zeros((16,), jnp.float32),
                                        unroll=8)
                    s = jnp.sum(acc)
                    return jnp.where(iota16 == r, s, h16)
                h16 = lax.fori_loop(0, 16, row_body,
                                    jnp.zeros((16,), jnp.float32))
                j = c * (_CH // 16) + rg
                g_v[pl.ds(j * 16, 16)] = (
                    _gelu16(h16) * w_v[ti, pl.ds(j * 16, 16)])

        cp_a.wait()
        down_chunk(0, rows_a)
        cp_b.wait()
        up_a = pltpu.make_async_copy(
            up_hbm.at[idx_v.at[ti, pl.ds(0, _CH)]], rows_a, sem_a)
        up_a.start()
        down_chunk(1, rows_b)
        up_b = pltpu.make_async_copy(
            up_hbm.at[idx_v.at[ti, pl.ds(_CH, _CH)]], rows_b, sem_b)
        up_b.start()

        def zero_body(cc, _):
            out_v[pl.ds(cc * 16, 16)] = jnp.zeros((16,), jnp.float32)
            return 0
        lax.fori_loop(0, N_DIM // 16, zero_body, 0)

        def up_chunk(c, rows_v):
            def urow_body(r, _):
                rg16 = g_v[pl.ds(c * _CH + (r // 16) * 16, 16)]
                s = jnp.sum(jnp.where(iota16 == (r % 16), rg16, 0.0))
                gb = lax.broadcast(s, (16,))

                def chunk_body(cc, _):
                    sl = pl.ds(cc * 16, 16)
                    out_v[sl] = out_v[sl] + rows_v[r, sl] * gb
                    return 0
                lax.fori_loop(0, N_DIM // 16, chunk_body, 0, unroll=8)
                return 0
            lax.fori_loop(0, _CH, urow_body, 0)

        up_a.wait()
        up_chunk(0, rows_a)
        up_b.wait()
        up_chunk(1, rows_b)

        pltpu.sync_copy(out_v, out_hbm.at[t])
        return carry

    lax.fori_loop(0, _TPW, token_body, 0)


def _sc_combine(x2d, idx, w, down_embed, up_embed):
    mesh = plsc.VectorSubcoreMesh(core_axis_name="c", subcore_axis_name="s")
    f = pl.kernel(
        _sc_body,
        out_type=jax.ShapeDtypeStruct((N_TOKENS, N_DIM), jnp.float32),
        mesh=mesh,
        compiler_params=pltpu.CompilerParams(needs_layout_passes=False),
        scratch_types=[
            pltpu.VMEM((N_DIM,), jnp.float32),
            pltpu.VMEM((_TPW, PICKS), jnp.int32),
            pltpu.VMEM((_TPW, PICKS), jnp.float32),
            pltpu.VMEM((PICKS,), jnp.float32),
            pltpu.VMEM((_CH, N_DIM), jnp.float32),
            pltpu.VMEM((_CH, N_DIM), jnp.float32),
            pltpu.VMEM((N_DIM,), jnp.float32),
            pltpu.SemaphoreType.DMA,
            pltpu.SemaphoreType.DMA,
        ],
    )
    return f(x2d, idx, w, down_embed, up_embed)


def kernel(x, W_q, keys, down_embed, up_embed):
    b, n, d = x.shape
    x2d = x.reshape(n, d)
    keys_t = jnp.transpose(keys, (2, 0, 1, 3))
    idx, w = _route(x2d.astype(jnp.bfloat16), W_q.astype(jnp.bfloat16),
                    keys_t.astype(jnp.bfloat16))
    idx = idx.reshape(N_TOKENS, PICKS)
    w = w.reshape(N_TOKENS, PICKS)
    out = _sc_combine(x2d, idx, w, down_embed, up_embed)
    return out.reshape(b, n, d)
